# baseline (device time: 204307 ns/iter reference)
import jax
import jax.numpy as jnp
from jax import lax
from jax.experimental import pallas as pl
from jax.experimental.pallas import tpu as pltpu

N_DEV = 16


def kernel(A, B):
    m, k = A.shape
    _, n = B.shape

    def body(a_ref, b_ref, out_ref, comm_ref, send_sems, recv_sems):
        my = lax.axis_index("i")
        left = lax.rem(my + (N_DEV - 1), N_DEV)
        right = lax.rem(my + 1, N_DEV)

        barrier_sem = pltpu.get_barrier_semaphore()
        for nbr in (left, right):
            pl.semaphore_signal(
                barrier_sem, inc=1,
                device_id=(nbr,), device_id_type=pl.DeviceIdType.MESH,
            )
        pl.semaphore_wait(barrier_sem, 2)

        partial = jnp.dot(a_ref[...], b_ref[...],
                          preferred_element_type=jnp.float32)
        out_ref[...] = partial
        comm_ref[0] = partial

        for h in range(N_DEV - 1):
            send_slot = h % 2
            recv_slot = (h + 1) % 2
            rdma = pltpu.make_async_remote_copy(
                src_ref=comm_ref.at[send_slot],
                dst_ref=comm_ref.at[recv_slot],
                send_sem=send_sems.at[send_slot],
                recv_sem=recv_sems.at[recv_slot],
                device_id=(right,),
                device_id_type=pl.DeviceIdType.MESH,
            )
            rdma.start()
            rdma.wait()
            out_ref[...] += comm_ref[recv_slot]

    return pl.pallas_call(
        body,
        out_shape=jax.ShapeDtypeStruct((m, n), jnp.float32),
        in_specs=[
            pl.BlockSpec(memory_space=pltpu.VMEM),
            pl.BlockSpec(memory_space=pltpu.VMEM),
        ],
        out_specs=pl.BlockSpec(memory_space=pltpu.VMEM),
        scratch_shapes=[
            pltpu.VMEM((2, m, n), jnp.float32),
            pltpu.SemaphoreType.DMA((2,)),
            pltpu.SemaphoreType.DMA((2,)),
        ],
        compiler_params=pltpu.CompilerParams(collective_id=0),
    )(A, B)


# device time: 43572 ns/iter; 4.6890x vs baseline; 4.6890x over previous
import jax
import jax.numpy as jnp
from jax import lax
from jax.experimental import pallas as pl
from jax.experimental.pallas import tpu as pltpu

N_DEV = 16
M = 512
RS_DIMS = (0, 2, 1, 3)
RS_HALVES = (M // 2, M // 4, M // 8, M // 16)


def kernel(A, B):
    m, k = A.shape
    _, n = B.shape

    def body(a_ref, b_ref, out_ref, acc_ref, rbuf0, rbuf1, rbuf2, rbuf3,
             rs_send, rs_recv, ag_send, ag_recv):
        my = lax.axis_index("i")
        rbufs = (rbuf0, rbuf1, rbuf2, rbuf3)

        barrier_sem = pltpu.get_barrier_semaphore()
        for d in RS_DIMS:
            partner = jnp.bitwise_xor(my, 1 << d)
            pl.semaphore_signal(
                barrier_sem, inc=1,
                device_id=(partner,), device_id_type=pl.DeviceIdType.MESH,
            )
        pl.semaphore_wait(barrier_sem, 4)

        acc_ref[...] = jnp.dot(a_ref[...], b_ref[...],
                               preferred_element_type=jnp.float32)

        start = jnp.int32(0)
        for s, d in enumerate(RS_DIMS):
            half = RS_HALVES[s]
            bit = jnp.bitwise_and(jnp.right_shift(my, d), 1)
            partner = jnp.bitwise_xor(my, 1 << d)
            send_start = start + (1 - bit) * half
            rdma = pltpu.make_async_remote_copy(
                src_ref=acc_ref.at[pl.ds(send_start, half)],
                dst_ref=rbufs[s],
                send_sem=rs_send.at[s],
                recv_sem=rs_recv.at[s],
                device_id=(partner,),
                device_id_type=pl.DeviceIdType.MESH,
            )
            rdma.start()
            rdma.wait()
            start = start + bit * half
            acc_ref[pl.ds(start, half), :] = (
                acc_ref[pl.ds(start, half), :] + rbufs[s][...]
            )

        seg = M // N_DEV
        out_ref[pl.ds(start, seg), :] = acc_ref[pl.ds(start, seg), :]

        size = seg
        for t, d in enumerate(reversed(RS_DIMS)):
            bit = jnp.bitwise_and(jnp.right_shift(my, d), 1)
            partner = jnp.bitwise_xor(my, 1 << d)
            rdma = pltpu.make_async_remote_copy(
                src_ref=out_ref.at[pl.ds(start, size)],
                dst_ref=out_ref.at[pl.ds(start, size)],
                send_sem=ag_send.at[t],
                recv_sem=ag_recv.at[t],
                device_id=(partner,),
                device_id_type=pl.DeviceIdType.MESH,
            )
            rdma.start()
            rdma.wait()
            start = start - bit * size
            size = size * 2

    return pl.pallas_call(
        body,
        out_shape=jax.ShapeDtypeStruct((m, n), jnp.float32),
        in_specs=[
            pl.BlockSpec(memory_space=pltpu.VMEM),
            pl.BlockSpec(memory_space=pltpu.VMEM),
        ],
        out_specs=pl.BlockSpec(memory_space=pltpu.VMEM),
        scratch_shapes=[
            pltpu.VMEM((m, n), jnp.float32),
            pltpu.VMEM((RS_HALVES[0], n), jnp.float32),
            pltpu.VMEM((RS_HALVES[1], n), jnp.float32),
            pltpu.VMEM((RS_HALVES[2], n), jnp.float32),
            pltpu.VMEM((RS_HALVES[3], n), jnp.float32),
            pltpu.SemaphoreType.DMA((4,)),
            pltpu.SemaphoreType.DMA((4,)),
            pltpu.SemaphoreType.DMA((4,)),
            pltpu.SemaphoreType.DMA((4,)),
        ],
        compiler_params=pltpu.CompilerParams(collective_id=0),
    )(A, B)


# device time: 35164 ns/iter; 5.8101x vs baseline; 1.2391x over previous
import jax
import jax.numpy as jnp
from jax import lax
from jax.experimental import pallas as pl
from jax.experimental.pallas import tpu as pltpu

N_DEV = 16
M = 512

GROUPS = ((0, 2), (1, 3))
MASKS = tuple(
    (1 << da, 1 << db, (1 << da) | (1 << db)) for (da, db) in GROUPS
)
ALL_MASKS = MASKS[0] + MASKS[1]
RS_QSIZE = (M // 4, M // 16)


def kernel(A, B):
    m, k = A.shape
    _, n = B.shape

    def body(a_ref, b_ref, out_ref, acc_ref, rbufA, rbufB,
             rs_send, rs_recv, ag_send, ag_recv):
        my = lax.axis_index("i")
        rbufs = (rbufA, rbufB)
        pending_sends = []

        barrier_sem = pltpu.get_barrier_semaphore()
        for mask in ALL_MASKS:
            pl.semaphore_signal(
                barrier_sem, inc=1,
                device_id=(jnp.bitwise_xor(my, mask),),
                device_id_type=pl.DeviceIdType.MESH,
            )
        pl.semaphore_wait(barrier_sem, len(ALL_MASKS))

        acc_ref[...] = jnp.dot(a_ref[...], b_ref[...],
                               preferred_element_type=jnp.float32)

        start = jnp.int32(0)
        for s, (da, db) in enumerate(GROUPS):
            qsize = RS_QSIZE[s]
            bita = jnp.bitwise_and(jnp.right_shift(my, da), 1)
            bitb = jnp.bitwise_and(jnp.right_shift(my, db), 1)
            qmine = bita + 2 * bitb
            rdmas = []
            for j, mask in enumerate(MASKS[s]):
                qp = (jnp.bitwise_xor(bita, (mask >> da) & 1)
                      + 2 * jnp.bitwise_xor(bitb, (mask >> db) & 1))
                rdma = pltpu.make_async_remote_copy(
                    src_ref=acc_ref.at[pl.ds(start + qp * qsize, qsize)],
                    dst_ref=rbufs[s].at[j],
                    send_sem=rs_send.at[3 * s + j],
                    recv_sem=rs_recv.at[3 * s + j],
                    device_id=(jnp.bitwise_xor(my, mask),),
                    device_id_type=pl.DeviceIdType.MESH,
                )
                rdma.start()
                rdmas.append(rdma)
            start = start + qmine * qsize
            for j, rdma in enumerate(rdmas):
                rdma.wait_recv()
                acc_ref[pl.ds(start, qsize), :] = (
                    acc_ref[pl.ds(start, qsize), :] + rbufs[s][j]
                )
            pending_sends.extend(rdmas)

        seg = M // N_DEV
        out_ref[pl.ds(start, seg), :] = acc_ref[pl.ds(start, seg), :]

        size = seg
        for t, (da, db) in enumerate(reversed(GROUPS)):
            s = len(GROUPS) - 1 - t
            bita = jnp.bitwise_and(jnp.right_shift(my, da), 1)
            bitb = jnp.bitwise_and(jnp.right_shift(my, db), 1)
            qmine = bita + 2 * bitb
            rdmas = []
            for j, mask in enumerate(MASKS[s]):
                rdma = pltpu.make_async_remote_copy(
                    src_ref=out_ref.at[pl.ds(start, size)],
                    dst_ref=out_ref.at[pl.ds(start, size)],
                    send_sem=ag_send.at[3 * s + j],
                    recv_sem=ag_recv.at[3 * s + j],
                    device_id=(jnp.bitwise_xor(my, mask),),
                    device_id_type=pl.DeviceIdType.MESH,
                )
                rdma.start()
                rdmas.append(rdma)
            for rdma in rdmas:
                rdma.wait_recv()
            pending_sends.extend(rdmas)
            start = start - qmine * size
            size = size * 4

        for rdma in pending_sends:
            rdma.wait_send()

    return pl.pallas_call(
        body,
        out_shape=jax.ShapeDtypeStruct((m, n), jnp.float32),
        in_specs=[
            pl.BlockSpec(memory_space=pltpu.VMEM),
            pl.BlockSpec(memory_space=pltpu.VMEM),
        ],
        out_specs=pl.BlockSpec(memory_space=pltpu.VMEM),
        scratch_shapes=[
            pltpu.VMEM((m, n), jnp.float32),
            pltpu.VMEM((3, RS_QSIZE[0], n), jnp.float32),
            pltpu.VMEM((3, RS_QSIZE[1], n), jnp.float32),
            pltpu.SemaphoreType.DMA((6,)),
            pltpu.SemaphoreType.DMA((6,)),
            pltpu.SemaphoreType.DMA((6,)),
            pltpu.SemaphoreType.DMA((6,)),
        ],
        compiler_params=pltpu.CompilerParams(collective_id=0),
    )(A, B)
